# baseline (speedup 1.0000x reference)
import functools

import jax
import jax.numpy as jnp
from jax import lax
from jax.experimental import pallas as pl
from jax.experimental.pallas import tpu as pltpu
from jax.experimental.pallas import tpu_sc as plsc

N = 16384
F = 1024
H1 = 512
H2 = 512
E = 8

BLK = 1024
NB = 23
NPAD = NB * BLK

NC = 2
NS = 16
NW = NC * NS
ROWS_PER_W = N // NW
CHUNK = 32
NCHUNK = ROWS_PER_W // CHUNK
GROUPS = CHUNK * F // 32


def _routing(species):
    s = species.astype(jnp.int32)
    eye = jnp.arange(E, dtype=jnp.int32)
    onehot = (s[:, None] == eye[None, :]).astype(jnp.int32)
    cum = jnp.cumsum(onehot, axis=0)
    counts = cum[-1]
    rank = jnp.sum(cum * onehot, axis=1) - 1
    padded = ((counts + BLK - 1) // BLK) * BLK
    pad_starts = jnp.concatenate(
        [jnp.zeros((1,), jnp.int32), jnp.cumsum(padded)[:-1].astype(jnp.int32)])
    dst = jnp.sum(onehot * pad_starts[None, :], axis=1) + rank
    b_idx = jnp.arange(NB, dtype=jnp.int32)
    bs = pad_starts // BLK
    be = (pad_starts + padded) // BLK
    in_reg = (b_idx[:, None] >= bs[None, :]) & (b_idx[:, None] < be[None, :])
    block_expert = jnp.sum(jnp.where(in_reg, eye[None, :], 0), axis=1)
    return dst.astype(jnp.int32), block_expert.astype(jnp.int32)


def _sc_pack_scatter_rows(features, dst):
    mesh = plsc.VectorSubcoreMesh(core_axis_name="c", subcore_axis_name="s")
    dst3 = dst.reshape(NW, NCHUNK, CHUNK)

    @functools.partial(
        pl.kernel,
        out_type=jax.ShapeDtypeStruct((NPAD, F // 2), jnp.int32),
        mesh=mesh,
        scratch_types=[
            pltpu.VMEM((NCHUNK, CHUNK), jnp.int32),
            pltpu.VMEM((2, CHUNK, F), jnp.float32),
            pltpu.VMEM((3, CHUNK, F // 2), jnp.int32),
            pltpu.SemaphoreType.DMA,
            pltpu.SemaphoreType.DMA,
            pltpu.SemaphoreType.DMA,
            pltpu.SemaphoreType.DMA,
            pltpu.SemaphoreType.DMA,
        ],
        compiler_params=pltpu.CompilerParams(needs_layout_passes=False),
    )
    def k(feat_hbm, dst_hbm, out_hbm, idx_v, rows_v, xb_v, si0, si1, so0, so1, so2):
        wid = lax.axis_index("s") * NC + lax.axis_index("c")
        sin = (si0, si1)
        sout = (so0, so1, so2)
        pltpu.sync_copy(dst_hbm.at[wid], idx_v)

        def start_in(c):
            base = wid * ROWS_PER_W + c * CHUNK
            return pltpu.async_copy(
                feat_hbm.at[pl.ds(base, CHUNK)], rows_v.at[c % 2], sin[c % 2])

        cps_in = [None] * NCHUNK
        cps_out = [None] * NCHUNK
        cps_in[0] = start_in(0)
        for c in range(NCHUNK):
            b = c % 2
            bo = c % 3
            cps_in[c].wait()
            if c + 1 < NCHUNK:
                cps_in[c + 1] = start_in(c + 1)
            if c >= 3:
                cps_out[c - 3].wait()

            @plsc.parallel_loop(0, GROUPS, unroll=16)
            def _(g):
                r = g // 32
                j0 = (g % 32) * 16
                a = rows_v[b, r, pl.ds(j0, 16)]
                bb = rows_v[b, r, pl.ds(j0 + F // 2, 16)]
                w = plsc.bitcast(
                    plsc.pack(a, bb, format=plsc.PackFormat.INTERLEAVED),
                    jnp.int32)
                xb_v[bo, r, pl.ds(j0, 16)] = w

            cps_out[c] = pltpu.async_copy(
                xb_v.at[bo], out_hbm.at[idx_v.at[c]], sout[bo])
        cps_out[NCHUNK - 3].wait()
        cps_out[NCHUNK - 2].wait()
        cps_out[NCHUNK - 1].wait()

    return k(features, dst3)


def _mlp_body(eid_ref, x_ref, w1a_ref, w1b_ref, b1_ref, w2_ref, b2_ref,
              w3_ref, b3_ref, out_ref):
    x32 = x_ref[...]
    xe = lax.bitcast_convert_type(x32 << 16, jnp.float32)
    xo = lax.bitcast_convert_type(x32 & jnp.int32(-65536), jnp.float32)
    h = jnp.tanh(
        jnp.dot(xe, w1a_ref[0], preferred_element_type=jnp.float32)
        + jnp.dot(xo, w1b_ref[0], preferred_element_type=jnp.float32)
        + b1_ref[0])
    h = jnp.tanh(
        jnp.dot(h, w2_ref[0], preferred_element_type=jnp.float32) + b2_ref[0])
    e = jnp.sum(h * w3_ref[0], axis=1, keepdims=True) + b3_ref[0]
    out_ref[0] = e


def _tc_grouped_mlp(block_expert, xs, W1, b1, W2, b2, W3, b3, nb=NB):
    b1r = b1.reshape(E, 1, H1)
    b2r = b2.reshape(E, 1, H2)
    w3r = W3.reshape(E, H2).reshape(E, 1, H2)
    b3r = b3.reshape(E, 1, 1)
    grid_spec = pltpu.PrefetchScalarGridSpec(
        num_scalar_prefetch=1,
        grid=(nb,),
        in_specs=[
            pl.BlockSpec((BLK, F // 2), lambda i, eid: (i, 0)),
            pl.BlockSpec((1, F // 2, H1), lambda i, eid: (eid[i], 0, 0)),
            pl.BlockSpec((1, F // 2, H1), lambda i, eid: (eid[i], 1, 0)),
            pl.BlockSpec((1, 1, H1), lambda i, eid: (eid[i], 0, 0)),
            pl.BlockSpec((1, H1, H2), lambda i, eid: (eid[i], 0, 0)),
            pl.BlockSpec((1, 1, H2), lambda i, eid: (eid[i], 0, 0)),
            pl.BlockSpec((1, 1, H2), lambda i, eid: (eid[i], 0, 0)),
            pl.BlockSpec((1, 1, 1), lambda i, eid: (eid[i], 0, 0)),
        ],
        out_specs=pl.BlockSpec((1, BLK, 1), lambda i, eid: (i, 0, 0)),
    )
    out = pl.pallas_call(
        _mlp_body,
        grid_spec=grid_spec,
        out_shape=jax.ShapeDtypeStruct((nb, BLK, 1), jnp.float32),
    )(block_expert, xs, W1, W1, b1r, W2, b2r, w3r, b3r)
    return out.reshape(nb * BLK)


def _sc_gather_out(e_pad, dst):
    mesh = plsc.VectorSubcoreMesh(core_axis_name="c", subcore_axis_name="s")

    @functools.partial(
        pl.kernel,
        out_type=jax.ShapeDtypeStruct((N,), jnp.float32),
        mesh=mesh,
        scratch_types=[
            pltpu.VMEM((NPAD,), jnp.float32),
            pltpu.VMEM((ROWS_PER_W,), jnp.int32),
            pltpu.VMEM((ROWS_PER_W,), jnp.float32),
        ],
        compiler_params=pltpu.CompilerParams(needs_layout_passes=False),
    )
    def k(e_hbm, dst_hbm, out_hbm, etab_v, idx_v, out_v):
        wid = lax.axis_index("s") * NC + lax.axis_index("c")
        base = wid * ROWS_PER_W
        pltpu.sync_copy(e_hbm, etab_v)
        pltpu.sync_copy(dst_hbm.at[pl.ds(base, ROWS_PER_W)], idx_v)
        for j in range(ROWS_PER_W // 16):
            idxs = idx_v[pl.ds(j * 16, 16)]
            out_v[pl.ds(j * 16, 16)] = plsc.load_gather(etab_v, [idxs])
        pltpu.sync_copy(out_v, out_hbm.at[pl.ds(base, ROWS_PER_W)])

    return k(e_pad, dst)


def kernel(features, species_indices, W1, b1, W2, b2, W3, b3):
    dst, block_expert = _routing(species_indices)
    xs32 = _sc_pack_scatter_rows(features, dst)
    e_pad = _tc_grouped_mlp(block_expert, xs32, W1, b1, W2, b2, W3, b3)
    return _sc_gather_out(e_pad, dst)


# speedup vs baseline: 1.0057x; 1.0057x over previous
import functools

import jax
import jax.numpy as jnp
from jax import lax
from jax.experimental import pallas as pl
from jax.experimental.pallas import tpu as pltpu
from jax.experimental.pallas import tpu_sc as plsc

N = 16384
F = 1024
H1 = 512
H2 = 512
E = 8

BLK = 1024
NB = 23
NPAD = NB * BLK

NC = 2
NS = 16
NW = NC * NS
ROWS_PER_W = N // NW
CHUNK = 32
NCHUNK = ROWS_PER_W // CHUNK
GROUPS = CHUNK * F // 32


def _routing(species):
    s = species.astype(jnp.int32)
    eye = jnp.arange(E, dtype=jnp.int32)
    onehot = (s[:, None] == eye[None, :]).astype(jnp.int32)
    cum = jnp.cumsum(onehot, axis=0)
    counts = cum[-1]
    rank = jnp.sum(cum * onehot, axis=1) - 1
    padded = ((counts + BLK - 1) // BLK) * BLK
    pad_starts = jnp.concatenate(
        [jnp.zeros((1,), jnp.int32), jnp.cumsum(padded)[:-1].astype(jnp.int32)])
    dst = jnp.sum(onehot * pad_starts[None, :], axis=1) + rank
    b_idx = jnp.arange(NB, dtype=jnp.int32)
    bs = pad_starts // BLK
    be = (pad_starts + padded) // BLK
    in_reg = (b_idx[:, None] >= bs[None, :]) & (b_idx[:, None] < be[None, :])
    block_expert = jnp.sum(jnp.where(in_reg, eye[None, :], 0), axis=1)
    return dst.astype(jnp.int32), block_expert.astype(jnp.int32)


def _sc_pack_scatter_rows(features, dst):
    mesh = plsc.VectorSubcoreMesh(core_axis_name="c", subcore_axis_name="s")
    dst3 = dst.reshape(NW, NCHUNK, CHUNK)

    @functools.partial(
        pl.kernel,
        out_type=jax.ShapeDtypeStruct((NPAD, F // 2), jnp.int32),
        mesh=mesh,
        scratch_types=[
            pltpu.VMEM((NCHUNK, CHUNK), jnp.int32),
            pltpu.VMEM((2, CHUNK, F), jnp.float32),
            pltpu.VMEM((2, CHUNK, F // 2), jnp.int32),
            pltpu.SemaphoreType.DMA,
            pltpu.SemaphoreType.DMA,
            pltpu.SemaphoreType.DMA,
            pltpu.SemaphoreType.DMA,
        ],
        compiler_params=pltpu.CompilerParams(needs_layout_passes=False),
    )
    def k(feat_hbm, dst_hbm, out_hbm, idx_v, rows_v, xb_v, si0, si1, so0, so1):
        wid = lax.axis_index("s") * NC + lax.axis_index("c")
        sin = (si0, si1)
        sout = (so0, so1)
        pltpu.sync_copy(dst_hbm.at[wid], idx_v)

        def start_in(c):
            base = wid * ROWS_PER_W + c * CHUNK
            return pltpu.async_copy(
                feat_hbm.at[pl.ds(base, CHUNK)], rows_v.at[c % 2], sin[c % 2])

        cps_in = [None] * NCHUNK
        cps_out = [None] * NCHUNK
        cps_in[0] = start_in(0)
        for c in range(NCHUNK):
            b = c % 2
            cps_in[c].wait()
            if c + 1 < NCHUNK:
                cps_in[c + 1] = start_in(c + 1)
            if c >= 2:
                cps_out[c - 2].wait()

            @plsc.parallel_loop(0, GROUPS, unroll=8)
            def _(g):
                r = g // 32
                j0 = (g % 32) * 16
                a = rows_v[b, r, pl.ds(j0, 16)]
                bb = rows_v[b, r, pl.ds(j0 + F // 2, 16)]
                w = plsc.bitcast(
                    plsc.pack(a, bb, format=plsc.PackFormat.INTERLEAVED),
                    jnp.int32)
                xb_v[b, r, pl.ds(j0, 16)] = w

            cps_out[c] = pltpu.async_copy(
                xb_v.at[b], out_hbm.at[idx_v.at[c]], sout[b])
        cps_out[NCHUNK - 2].wait()
        cps_out[NCHUNK - 1].wait()

    return k(features, dst3)


def _mlp_body(eid_ref, x_ref, w1a_ref, w1b_ref, b1_ref, w2_ref, b2_ref,
              w3_ref, b3_ref, out_ref):
    x32 = x_ref[...]
    xe = lax.bitcast_convert_type(x32 << 16, jnp.float32)
    xo = lax.bitcast_convert_type(x32 & jnp.int32(-65536), jnp.float32)
    h = jnp.tanh(
        jnp.dot(xe, w1a_ref[0], preferred_element_type=jnp.float32)
        + jnp.dot(xo, w1b_ref[0], preferred_element_type=jnp.float32)
        + b1_ref[0])
    h = jnp.tanh(
        jnp.dot(h, w2_ref[0], preferred_element_type=jnp.float32) + b2_ref[0])
    e = jnp.sum(h * w3_ref[0], axis=1, keepdims=True) + b3_ref[0]
    out_ref[0] = e


def _tc_grouped_mlp(block_expert, xs, W1, b1, W2, b2, W3, b3, nb=NB):
    b1r = b1.reshape(E, 1, H1)
    b2r = b2.reshape(E, 1, H2)
    w3r = W3.reshape(E, H2).reshape(E, 1, H2)
    b3r = b3.reshape(E, 1, 1)
    grid_spec = pltpu.PrefetchScalarGridSpec(
        num_scalar_prefetch=1,
        grid=(nb,),
        in_specs=[
            pl.BlockSpec((BLK, F // 2), lambda i, eid: (i, 0)),
            pl.BlockSpec((1, F // 2, H1), lambda i, eid: (eid[i], 0, 0)),
            pl.BlockSpec((1, F // 2, H1), lambda i, eid: (eid[i], 1, 0)),
            pl.BlockSpec((1, 1, H1), lambda i, eid: (eid[i], 0, 0)),
            pl.BlockSpec((1, H1, H2), lambda i, eid: (eid[i], 0, 0)),
            pl.BlockSpec((1, 1, H2), lambda i, eid: (eid[i], 0, 0)),
            pl.BlockSpec((1, 1, H2), lambda i, eid: (eid[i], 0, 0)),
            pl.BlockSpec((1, 1, 1), lambda i, eid: (eid[i], 0, 0)),
        ],
        out_specs=pl.BlockSpec((1, BLK, 1), lambda i, eid: (i, 0, 0)),
    )
    out = pl.pallas_call(
        _mlp_body,
        grid_spec=grid_spec,
        out_shape=jax.ShapeDtypeStruct((nb, BLK, 1), jnp.float32),
    )(block_expert, xs, W1, W1, b1r, W2, b2r, w3r, b3r)
    return out.reshape(nb * BLK)


def _sc_gather_out(e_pad, dst):
    mesh = plsc.VectorSubcoreMesh(core_axis_name="c", subcore_axis_name="s")

    @functools.partial(
        pl.kernel,
        out_type=jax.ShapeDtypeStruct((N,), jnp.float32),
        mesh=mesh,
        scratch_types=[
            pltpu.VMEM((NPAD,), jnp.float32),
            pltpu.VMEM((ROWS_PER_W,), jnp.int32),
            pltpu.VMEM((ROWS_PER_W,), jnp.float32),
        ],
        compiler_params=pltpu.CompilerParams(needs_layout_passes=False),
    )
    def k(e_hbm, dst_hbm, out_hbm, etab_v, idx_v, out_v):
        wid = lax.axis_index("s") * NC + lax.axis_index("c")
        base = wid * ROWS_PER_W
        pltpu.sync_copy(e_hbm, etab_v)
        pltpu.sync_copy(dst_hbm.at[pl.ds(base, ROWS_PER_W)], idx_v)
        for j in range(ROWS_PER_W // 16):
            idxs = idx_v[pl.ds(j * 16, 16)]
            out_v[pl.ds(j * 16, 16)] = plsc.load_gather(etab_v, [idxs])
        pltpu.sync_copy(out_v, out_hbm.at[pl.ds(base, ROWS_PER_W)])

    return k(e_pad, dst)


def kernel(features, species_indices, W1, b1, W2, b2, W3, b3):
    dst, block_expert = _routing(species_indices)
    xs32 = _sc_pack_scatter_rows(features, dst)
    e_pad = _tc_grouped_mlp(block_expert, xs32, W1, b1, W2, b2, W3, b3)
    return _sc_gather_out(e_pad, dst)
